# baseline (speedup 1.0000x reference)
import functools

import jax
import jax.numpy as jnp
from jax import lax
from jax.experimental import pallas as pl
from jax.experimental.pallas import tpu as pltpu
from jax.experimental.pallas import tpu_sc as plsc

_NUM_CORES = 2
_NUM_SUBCORES = 16
_NUM_WORKERS = _NUM_CORES * _NUM_SUBCORES
_LANES = 16


@functools.lru_cache(maxsize=None)
def _build_gather(batch: int, hist: int, vocab: int, row_chunk: int):
    rows_per_w = batch // _NUM_WORKERS
    n_chunks = rows_per_w // row_chunk
    assert rows_per_w % row_chunk == 0
    n_full_win = hist // _LANES
    tail_w = hist - n_full_win * _LANES
    full = n_full_win * _LANES
    assert tail_w in (0, 8)
    fs = full + (_LANES if tail_w else 0)
    flat = row_chunk * fs
    mesh = plsc.VectorSubcoreMesh(core_axis_name="c", subcore_axis_name="s")

    piece = 12800
    n_full = vocab // piece
    tail = vocab - n_full * piece
    n_pieces = n_full + (1 if tail else 0)
    assert piece % 8 == 0 and tail % 8 == 0 and piece <= flat
    rounds = -(-n_pieces // _NUM_SUBCORES)

    @functools.partial(
        pl.kernel,
        out_type=jax.ShapeDtypeStruct((batch, hist), jnp.float32),
        mesh=mesh,
        scratch_types=[
            pltpu.VMEM((row_chunk, hist), jnp.int32),
            pltpu.VMEM((row_chunk, hist), jnp.float32),
            pltpu.VMEM((flat,), jnp.int32),
            pltpu.VMEM((flat,), jnp.float32),
            pltpu.VMEM_SHARED((vocab,), jnp.float32),
            pltpu.SemaphoreType.DMA,
            pltpu.SemaphoreType.DMA,
        ],
    )
    def gather(
        table_hbm,
        idx_hbm,
        out_hbm,
        idx2d,
        out2d,
        flat_idx,
        flat_rows,
        sh_table,
        sem_g,
        sem_st,
    ):
        sid = lax.axis_index("s")
        wid = sid * _NUM_CORES + lax.axis_index("c")
        row0 = wid * rows_per_w

        for r in range(rounds):
            p = sid + r * _NUM_SUBCORES

            @pl.when(p < n_full)
            def _():
                pltpu.sync_copy(
                    table_hbm.at[pl.ds(p * piece, piece)],
                    flat_rows.at[pl.ds(0, piece)],
                )
                pltpu.sync_copy(
                    flat_rows.at[pl.ds(0, piece)],
                    sh_table.at[pl.ds(p * piece, piece)],
                )

            if tail:

                @pl.when(p == n_full)
                def _():
                    pltpu.sync_copy(
                        table_hbm.at[pl.ds(n_full * piece, tail)],
                        flat_rows.at[pl.ds(0, tail)],
                    )
                    pltpu.sync_copy(
                        flat_rows.at[pl.ds(0, tail)],
                        sh_table.at[pl.ds(n_full * piece, tail)],
                    )

        plsc.subcore_barrier()

        for i in range(n_chunks):
            r0 = row0 + i * row_chunk
            pltpu.sync_copy(idx_hbm.at[pl.ds(r0, row_chunk), :], idx2d)

            def repack_in(j):
                fbase = pl.multiple_of(j * fs, _LANES)
                for w in range(n_full_win):
                    v = idx2d[j, pl.ds(w * _LANES, _LANES)]
                    flat_idx[pl.ds(fbase + w * _LANES, _LANES)] = v
                if tail_w:
                    v = idx2d[j, pl.ds(hist - _LANES, _LANES)]
                    flat_idx[pl.ds(fbase + full, _LANES)] = v

            pl.loop(0, row_chunk)(repack_in)
            pltpu.async_copy(sh_table.at[flat_idx], flat_rows, sem_g).wait()

            def repack_out(j):
                fbase = pl.multiple_of(j * fs, _LANES)
                if tail_w:
                    v = flat_rows[pl.ds(fbase + full, _LANES)]
                    out2d[j, pl.ds(hist - _LANES, _LANES)] = v
                for w in range(n_full_win):
                    v = flat_rows[pl.ds(fbase + w * _LANES, _LANES)]
                    out2d[j, pl.ds(w * _LANES, _LANES)] = v

            pl.loop(0, row_chunk)(repack_out)
            pltpu.sync_copy(out2d, out_hbm.at[pl.ds(r0, row_chunk), :])

    return gather


def kernel(table, user_ids, item_ids):
    del user_ids
    b, h = item_ids.shape
    return _build_gather(b, h, table.shape[0], 64)(table, item_ids)


# speedup vs baseline: 1.2972x; 1.2972x over previous
import functools

import jax
import jax.numpy as jnp
from jax import lax
from jax.experimental import pallas as pl
from jax.experimental.pallas import tpu as pltpu
from jax.experimental.pallas import tpu_sc as plsc

_NUM_CORES = 2
_NUM_SUBCORES = 16
_NUM_WORKERS = _NUM_CORES * _NUM_SUBCORES
_LANES = 16


@functools.lru_cache(maxsize=None)
def _build_gather(batch: int, hist: int, vocab: int, row_chunk: int):
    rows_per_w = batch // _NUM_WORKERS
    n_chunks = rows_per_w // row_chunk
    assert rows_per_w % row_chunk == 0
    n_full_win = hist // _LANES
    tail_w = hist - n_full_win * _LANES
    full = n_full_win * _LANES
    assert tail_w in (0, 8)
    fs = full + (_LANES if tail_w else 0)
    flat = row_chunk * fs
    mesh = plsc.VectorSubcoreMesh(core_axis_name="c", subcore_axis_name="s")

    piece = 12800
    n_full = vocab // piece
    tail = vocab - n_full * piece
    n_pieces = n_full + (1 if tail else 0)
    assert piece % 8 == 0 and tail % 8 == 0 and piece <= flat
    rounds = -(-n_pieces // _NUM_SUBCORES)

    @functools.partial(
        pl.kernel,
        out_type=jax.ShapeDtypeStruct((batch, hist), jnp.float32),
        mesh=mesh,
        scratch_types=[
            pltpu.VMEM((row_chunk, hist), jnp.int32),
            pltpu.VMEM((row_chunk, hist), jnp.float32),
            pltpu.VMEM((flat,), jnp.int32),
            pltpu.VMEM((flat,), jnp.float32),
            pltpu.VMEM_SHARED((vocab,), jnp.float32),
            pltpu.SemaphoreType.DMA,
            pltpu.SemaphoreType.DMA,
        ],
    )
    def gather(
        table_hbm,
        idx_hbm,
        out_hbm,
        idx2d,
        out2d,
        flat_idx,
        flat_rows,
        sh_table,
        sem_g,
        sem_st,
    ):
        sid = lax.axis_index("s")
        wid = sid * _NUM_CORES + lax.axis_index("c")
        row0 = wid * rows_per_w

        for r in range(rounds):
            p = sid + r * _NUM_SUBCORES

            @pl.when(p < n_full)
            def _():
                pltpu.sync_copy(
                    table_hbm.at[pl.ds(p * piece, piece)],
                    flat_rows.at[pl.ds(0, piece)],
                )
                pltpu.sync_copy(
                    flat_rows.at[pl.ds(0, piece)],
                    sh_table.at[pl.ds(p * piece, piece)],
                )

            if tail:

                @pl.when(p == n_full)
                def _():
                    pltpu.sync_copy(
                        table_hbm.at[pl.ds(n_full * piece, tail)],
                        flat_rows.at[pl.ds(0, tail)],
                    )
                    pltpu.sync_copy(
                        flat_rows.at[pl.ds(0, tail)],
                        sh_table.at[pl.ds(n_full * piece, tail)],
                    )

        plsc.subcore_barrier()

        for i in range(n_chunks):
            r0 = row0 + i * row_chunk
            pltpu.sync_copy(idx_hbm.at[pl.ds(r0, row_chunk), :], idx2d)

            def repack_in(j):
                fbase = pl.multiple_of(j * fs, _LANES)
                for w in range(n_full_win):
                    v = idx2d[j, pl.ds(w * _LANES, _LANES)]
                    flat_idx[pl.ds(fbase + w * _LANES, _LANES)] = v
                if tail_w:
                    v = idx2d[j, pl.ds(hist - _LANES, _LANES)]
                    flat_idx[pl.ds(fbase + full, _LANES)] = v

            plsc.parallel_loop(0, row_chunk, unroll=2)(repack_in)
            pltpu.async_copy(sh_table.at[flat_idx], flat_rows, sem_g).wait()

            def repack_out(j):
                fbase = pl.multiple_of(j * fs, _LANES)
                if tail_w:
                    v = flat_rows[pl.ds(fbase + full, _LANES)]
                    out2d[j, pl.ds(hist - _LANES, _LANES)] = v
                for w in range(n_full_win):
                    v = flat_rows[pl.ds(fbase + w * _LANES, _LANES)]
                    out2d[j, pl.ds(w * _LANES, _LANES)] = v

            plsc.parallel_loop(0, row_chunk, unroll=2)(repack_out)
            pltpu.sync_copy(out2d, out_hbm.at[pl.ds(r0, row_chunk), :])

    return gather


def kernel(table, user_ids, item_ids):
    del user_ids
    b, h = item_ids.shape
    return _build_gather(b, h, table.shape[0], 64)(table, item_ids)


# speedup vs baseline: 1.5822x; 1.2198x over previous
import functools

import jax
import jax.numpy as jnp
from jax import lax
from jax.experimental import pallas as pl
from jax.experimental.pallas import tpu as pltpu
from jax.experimental.pallas import tpu_sc as plsc

_NUM_CORES = 2
_NUM_SUBCORES = 16
_NUM_WORKERS = _NUM_CORES * _NUM_SUBCORES
_LANES = 16


@functools.lru_cache(maxsize=None)
def _build_gather(batch: int, hist: int, vocab: int, row_chunk: int):
    rows_per_w = batch // _NUM_WORKERS
    n_chunks = rows_per_w // row_chunk
    assert rows_per_w % row_chunk == 0 and n_chunks >= 2
    n_full_win = hist // _LANES
    tail_w = hist - n_full_win * _LANES
    full = n_full_win * _LANES
    assert tail_w in (0, 8)
    fs = full + (_LANES if tail_w else 0)
    flat = row_chunk * fs
    mesh = plsc.VectorSubcoreMesh(core_axis_name="c", subcore_axis_name="s")

    piece = 12800
    n_fullp = vocab // piece
    ptail = vocab - n_fullp * piece
    n_pieces = n_fullp + (1 if ptail else 0)
    assert piece % 8 == 0 and ptail % 8 == 0
    rounds = -(-n_pieces // _NUM_SUBCORES)

    @functools.partial(
        pl.kernel,
        out_type=jax.ShapeDtypeStruct((batch, hist), jnp.float32),
        mesh=mesh,
        scratch_types=[
            pltpu.VMEM((row_chunk, hist), jnp.int32),
            pltpu.VMEM((row_chunk, hist), jnp.float32),
            pltpu.VMEM((flat,), jnp.int32),
            pltpu.VMEM((flat,), jnp.int32),
            pltpu.VMEM((flat,), jnp.float32),
            pltpu.VMEM((flat,), jnp.float32),
            pltpu.VMEM((piece,), jnp.float32),
            pltpu.VMEM_SHARED((vocab,), jnp.float32),
            pltpu.SemaphoreType.DMA,
            pltpu.SemaphoreType.DMA,
            pltpu.SemaphoreType.DMA,
            pltpu.SemaphoreType.DMA,
        ],
    )
    def gather(
        table_hbm,
        idx_hbm,
        out_hbm,
        idx2d,
        out2d,
        fidx0,
        fidx1,
        frows0,
        frows1,
        stage_v,
        sh_table,
        sem_in,
        sem_out,
        sem_g0,
        sem_g1,
    ):
        sid = lax.axis_index("s")
        wid = sid * _NUM_CORES + lax.axis_index("c")
        row0 = wid * rows_per_w
        fidx = (fidx0, fidx1)
        frows = (frows0, frows1)
        sem_g = (sem_g0, sem_g1)

        def in_copy(i):
            return pltpu.make_async_copy(
                idx_hbm.at[pl.ds(row0 + i * row_chunk, row_chunk), :], idx2d, sem_in
            )

        def out_copy(i):
            return pltpu.make_async_copy(
                out2d, out_hbm.at[pl.ds(row0 + i * row_chunk, row_chunk), :], sem_out
            )

        def g_copy(i):
            b = i % 2
            return pltpu.make_async_copy(sh_table.at[fidx[b]], frows[b], sem_g[b])

        in_copy(0).start()

        for r in range(rounds):
            p = sid + r * _NUM_SUBCORES

            @pl.when(p < n_fullp)
            def _():
                pltpu.sync_copy(table_hbm.at[pl.ds(p * piece, piece)], stage_v)
                pltpu.sync_copy(stage_v, sh_table.at[pl.ds(p * piece, piece)])

            if ptail:

                @pl.when(p == n_fullp)
                def _():
                    pltpu.sync_copy(
                        table_hbm.at[pl.ds(n_fullp * piece, ptail)],
                        stage_v.at[pl.ds(0, ptail)],
                    )
                    pltpu.sync_copy(
                        stage_v.at[pl.ds(0, ptail)],
                        sh_table.at[pl.ds(n_fullp * piece, ptail)],
                    )

        plsc.subcore_barrier()

        def repack_in(b):
            def body(j):
                fbase = pl.multiple_of(j * fs, _LANES)
                for w in range(n_full_win):
                    v = idx2d[j, pl.ds(w * _LANES, _LANES)]
                    fidx[b][pl.ds(fbase + w * _LANES, _LANES)] = v
                if tail_w:
                    v = idx2d[j, pl.ds(hist - _LANES, _LANES)]
                    fidx[b][pl.ds(fbase + full, _LANES)] = v

            plsc.parallel_loop(0, row_chunk, unroll=2)(body)

        def repack_out(b):
            def body(j):
                fbase = pl.multiple_of(j * fs, _LANES)
                if tail_w:
                    v = frows[b][pl.ds(fbase + full, _LANES)]
                    out2d[j, pl.ds(hist - _LANES, _LANES)] = v
                for w in range(n_full_win):
                    v = frows[b][pl.ds(fbase + w * _LANES, _LANES)]
                    out2d[j, pl.ds(w * _LANES, _LANES)] = v

            plsc.parallel_loop(0, row_chunk, unroll=2)(body)

        for i in range(n_chunks):
            b = i % 2
            in_copy(i).wait()
            repack_in(b)
            g_copy(i).start()
            if i + 1 < n_chunks:
                in_copy(i + 1).start()
            if i >= 1:
                g_copy(i - 1).wait()
                if i >= 2:
                    out_copy(i - 2).wait()
                repack_out((i - 1) % 2)
                out_copy(i - 1).start()
        g_copy(n_chunks - 1).wait()
        out_copy(n_chunks - 2).wait()
        repack_out((n_chunks - 1) % 2)
        out_copy(n_chunks - 1).start()
        out_copy(n_chunks - 1).wait()

    return gather


def kernel(table, user_ids, item_ids):
    del user_ids
    b, h = item_ids.shape
    return _build_gather(b, h, table.shape[0], 32)(table, item_ids)
